# baseline (speedup 1.0000x reference)
import functools

import jax
import jax.numpy as jnp
from jax import lax
from jax.experimental import pallas as pl
from jax.experimental.pallas import tpu as pltpu
from jax.experimental.pallas import tpu_sc as plsc

B = 128
VOCAB = 100000
TOPK = 5
L = 16

SCROWS = 32
RSPLIT = B - SCROWS
CHUNKS = VOCAB // L
UNROLL = 25
OUTER = CHUNKS // UNROLL

RB = 8
NRB = RSPLIT // RB


def _sc_scan_body(yp_hbm, y_hbm, cnt_hbm, yv, row_v, cv, sem):
    c = lax.axis_index("c")
    s = lax.axis_index("s")
    wid = s * 2 + c
    b = RSPLIT + wid
    pltpu.sync_copy(y_hbm, yv)
    pltpu.sync_copy(yp_hbm.at[b], row_v)
    iota = lax.iota(jnp.int32, L)
    bsplat = jnp.zeros((L,), jnp.int32) + b
    yb = plsc.load_gather(yv, [bsplat])
    t = plsc.load_gather(row_v, [yb])

    def step(o, acc):
        base = o * (L * UNROLL)
        for u in range(UNROLL):
            cbase = base + u * L
            col = iota + cbase
            v = plsc.load_gather(row_v, [col])
            m = (v > t) | ((v == t) & (col < yb))
            acc = acc + plsc.all_reduce_population_count(m)
        return acc

    acc = lax.fori_loop(0, OUTER, step, jnp.zeros((L,), jnp.int32))
    cv[...] = acc
    pltpu.sync_copy(cv, cnt_hbm.at[wid])


@functools.cache
def _sc_scan():
    return pl.kernel(
        _sc_scan_body,
        out_type=jax.ShapeDtypeStruct((SCROWS, L), jnp.int32),
        mesh=plsc.VectorSubcoreMesh(core_axis_name="c", subcore_axis_name="s"),
        compiler_params=pltpu.CompilerParams(needs_layout_passes=False),
        scratch_types=[
            pltpu.VMEM((B,), jnp.int32),
            pltpu.VMEM((VOCAB,), jnp.float32),
            pltpu.VMEM((L,), jnp.int32),
            pltpu.SemaphoreType.DMA,
        ],
    )


def _tc_scan_body(y_ref, cnt_ref, x_ref, out_ref, acc_ref):
    i = pl.program_id(0)
    yy = y_ref[...]
    vals = x_ref[...]
    col = lax.broadcasted_iota(jnp.int32, (RB, VOCAB), 1)
    t = jnp.sum(jnp.where(col == yy, vals, 0.0), axis=1, keepdims=True)
    m = (vals > t) | ((vals == t) & (col < yy))
    acc_ref[pl.ds(i * RB, RB), :] = jnp.sum(
        m.astype(jnp.int32), axis=1, keepdims=True
    )

    @pl.when(i == NRB - 1)
    def _():
        tc_hits = jnp.sum(
            (acc_ref[...] < TOPK).astype(jnp.int32), axis=(0, 1), keepdims=True
        )
        sc_hits = jnp.sum(
            (cnt_ref[...][:, :1] < TOPK).astype(jnp.int32),
            axis=(0, 1),
            keepdims=True,
        )
        out_ref[...] = tc_hits + sc_hits


def _tc_scan(y_pred, y, cnt):
    return pl.pallas_call(
        _tc_scan_body,
        grid=(NRB,),
        in_specs=[
            pl.BlockSpec((RB, 1), lambda i: (i, 0)),
            pl.BlockSpec((SCROWS, L), lambda i: (0, 0)),
            pl.BlockSpec((RB, VOCAB), lambda i: (i, 0)),
        ],
        out_specs=pl.BlockSpec((1, 1), lambda i: (0, 0)),
        out_shape=jax.ShapeDtypeStruct((1, 1), jnp.int32),
        scratch_shapes=[
            pltpu.VMEM((RSPLIT, 1), jnp.int32),
        ],
    )(y[:RSPLIT].reshape(RSPLIT, 1), cnt, y_pred)


def kernel(y_pred, y):
    y32 = y.astype(jnp.int32)
    cnt = _sc_scan()(y_pred, y32)
    return _tc_scan(y_pred, y32, cnt)[0, 0]


# speedup vs baseline: 1.2548x; 1.2548x over previous
import functools

import jax
import jax.numpy as jnp
from jax import lax
from jax.experimental import pallas as pl
from jax.experimental.pallas import tpu as pltpu
from jax.experimental.pallas import tpu_sc as plsc

B = 128
VOCAB = 100000
TOPK = 5
LANES = 16

RB = 8
NRB = B // RB


H = B // 2


def _gather_body(yp_hbm, y_hbm, g_hbm, ys, buf, sem):
    c = lax.axis_index("c")

    def half(lo):
        pltpu.sync_copy(y_hbm, ys)
        copies = []
        for r in range(H):
            start = (ys[lo + r] // LANES) * LANES
            copies.append(
                pltpu.async_copy(
                    yp_hbm.at[lo + r, pl.ds(start, LANES)], buf.at[r], sem
                )
            )
        for cp in copies:
            cp.wait()
        pltpu.sync_copy(buf, g_hbm.at[pl.ds(lo, H)])

    @pl.when(c == 0)
    def _():
        half(0)

    @pl.when(c == 1)
    def _():
        half(H)


@functools.cache
def _gather_segments():
    return pl.kernel(
        _gather_body,
        out_type=jax.ShapeDtypeStruct((B, LANES), jnp.float32),
        mesh=plsc.ScalarSubcoreMesh(axis_name="c", num_cores=2),
        scratch_types=[
            pltpu.SMEM((B,), jnp.int32),
            pltpu.VMEM_SHARED((H, LANES), jnp.float32),
            pltpu.SemaphoreType.DMA,
        ],
    )


def _scan_body(g_ref, y_ref, x_ref, out_ref, acc_ref):
    i = pl.program_id(0)
    yy = y_ref[...]
    seg_lane = lax.broadcasted_iota(jnp.int32, (RB, LANES), 1)
    t = jnp.sum(
        jnp.where(seg_lane == yy % LANES, g_ref[...], 0.0),
        axis=1,
        keepdims=True,
    )
    vals = x_ref[...]
    col = lax.broadcasted_iota(jnp.int32, (RB, VOCAB), 1)
    m = (vals > t) | ((vals == t) & (col < yy))
    acc_ref[pl.ds(i * RB, RB), :] = jnp.sum(
        m.astype(jnp.int32), axis=1, keepdims=True
    )

    @pl.when(i == NRB - 1)
    def _():
        out_ref[...] = jnp.sum(
            (acc_ref[...] < TOPK).astype(jnp.int32), axis=(0, 1), keepdims=True
        )


def _count_hits(y_pred, g, y):
    return pl.pallas_call(
        _scan_body,
        grid=(NRB,),
        in_specs=[
            pl.BlockSpec((RB, LANES), lambda i: (i, 0)),
            pl.BlockSpec((RB, 1), lambda i: (i, 0)),
            pl.BlockSpec((RB, VOCAB), lambda i: (i, 0)),
        ],
        out_specs=pl.BlockSpec((1, 1), lambda i: (0, 0)),
        out_shape=jax.ShapeDtypeStruct((1, 1), jnp.int32),
        scratch_shapes=[
            pltpu.VMEM((B, 1), jnp.int32),
        ],
    )(g, y.reshape(B, 1), y_pred)


def kernel(y_pred, y):
    y32 = y.astype(jnp.int32)
    g = _gather_segments()(y_pred, y32)
    return _count_hits(y_pred, g, y32)[0, 0]


# speedup vs baseline: 1.3207x; 1.0525x over previous
import functools

import jax
import jax.numpy as jnp
from jax import lax
from jax.experimental import pallas as pl
from jax.experimental.pallas import tpu as pltpu
from jax.experimental.pallas import tpu_sc as plsc

B = 128
VOCAB = 100000
TOPK = 5
LANES = 16

RB = 8
NRB = B // RB


H = B // 2


def _gather_body(yp_hbm, y_hbm, g_hbm, ys, buf, sem):
    c = lax.axis_index("c")

    def half(lo):
        pltpu.sync_copy(y_hbm, ys)
        copies = []
        for r in range(H):
            start = (ys[lo + r] // LANES) * LANES
            copies.append(
                pltpu.async_copy(
                    yp_hbm.at[lo + r, pl.ds(start, LANES)], buf.at[r], sem
                )
            )
        for cp in copies:
            cp.wait()
        pltpu.sync_copy(buf, g_hbm.at[pl.ds(lo, H)])

    @pl.when(c == 0)
    def _():
        half(0)

    @pl.when(c == 1)
    def _():
        half(H)


@functools.cache
def _gather_segments():
    return pl.kernel(
        _gather_body,
        out_type=jax.ShapeDtypeStruct((B, LANES), jnp.float32),
        mesh=plsc.ScalarSubcoreMesh(axis_name="c", num_cores=2),
        scratch_types=[
            pltpu.SMEM((B,), jnp.int32),
            pltpu.VMEM_SHARED((H, LANES), jnp.float32),
            pltpu.SemaphoreType.DMA,
        ],
    )


def _scan_body(
    g0_ref, g1_ref, y0_ref, y1_ref, x0_ref, x1_ref, out_ref, acc_ref
):
    i = pl.program_id(0)
    seg_lane = lax.broadcasted_iota(jnp.int32, (RB, LANES), 1)
    col = lax.broadcasted_iota(jnp.int32, (RB, VOCAB), 1)
    for k, (g_ref, y_ref, x_ref) in enumerate(
        ((g0_ref, y0_ref, x0_ref), (g1_ref, y1_ref, x1_ref))
    ):
        yy = y_ref[...]
        t = jnp.sum(
            jnp.where(seg_lane == yy % LANES, g_ref[...], 0.0),
            axis=1,
            keepdims=True,
        )
        vals = x_ref[...]
        m = (vals > t) | ((vals == t) & (col < yy))
        acc_ref[pl.ds((2 * i + k) * RB, RB), :] = jnp.sum(
            m.astype(jnp.int32), axis=1, keepdims=True
        )

    @pl.when(i == NRB // 2 - 1)
    def _():
        out_ref[...] = jnp.sum(
            (acc_ref[...] < TOPK).astype(jnp.int32), axis=(0, 1), keepdims=True
        )


def _count_hits(y_pred, g, y):
    y2 = y.reshape(B, 1)
    return pl.pallas_call(
        _scan_body,
        grid=(NRB // 2,),
        in_specs=[
            pl.BlockSpec((RB, LANES), lambda i: (2 * i, 0)),
            pl.BlockSpec((RB, LANES), lambda i: (2 * i + 1, 0)),
            pl.BlockSpec((RB, 1), lambda i: (2 * i, 0)),
            pl.BlockSpec((RB, 1), lambda i: (2 * i + 1, 0)),
            pl.BlockSpec((RB, VOCAB), lambda i: (2 * i, 0)),
            pl.BlockSpec((RB, VOCAB), lambda i: (2 * i + 1, 0)),
        ],
        out_specs=pl.BlockSpec((1, 1), lambda i: (0, 0)),
        out_shape=jax.ShapeDtypeStruct((1, 1), jnp.int32),
        scratch_shapes=[
            pltpu.VMEM((B, 1), jnp.int32),
        ],
    )(g, g, y2, y2, y_pred, y_pred)


def kernel(y_pred, y):
    y32 = y.astype(jnp.int32)
    g = _gather_segments()(y_pred, y32)
    return _count_hits(y_pred, g, y32)[0, 0]


# speedup vs baseline: 1.3324x; 1.0089x over previous
import functools

import jax
import jax.numpy as jnp
from jax import lax
from jax.experimental import pallas as pl
from jax.experimental.pallas import tpu as pltpu
from jax.experimental.pallas import tpu_sc as plsc

B = 128
VOCAB = 100000
TOPK = 5
LANES = 16

RB = 8
NRB = B // RB


H = B // 2


def _gather_body(yp_hbm, y_hbm, g_hbm, ys, buf, sem):
    c = lax.axis_index("c")

    def half(lo):
        pltpu.sync_copy(y_hbm, ys)
        copies = []
        for r in range(H):
            start = (ys[lo + r] // LANES) * LANES
            copies.append(
                pltpu.async_copy(
                    yp_hbm.at[lo + r, pl.ds(start, LANES)], buf.at[r], sem
                )
            )
        for cp in copies:
            cp.wait()
        pltpu.sync_copy(buf, g_hbm.at[pl.ds(lo, H)])

    @pl.when(c == 0)
    def _():
        half(0)

    @pl.when(c == 1)
    def _():
        half(H)


@functools.cache
def _gather_segments():
    return pl.kernel(
        _gather_body,
        out_type=jax.ShapeDtypeStruct((B, LANES), jnp.float32),
        mesh=plsc.ScalarSubcoreMesh(axis_name="c", num_cores=2),
        scratch_types=[
            pltpu.SMEM((B,), jnp.int32),
            pltpu.VMEM_SHARED((H, LANES), jnp.float32),
            pltpu.SemaphoreType.DMA,
        ],
    )


NS = 4


def _scan_body(*refs):
    gs = refs[:NS]
    ys = refs[NS : 2 * NS]
    xs = refs[2 * NS : 3 * NS]
    out_ref = refs[3 * NS]
    acc_ref = refs[3 * NS + 1]
    i = pl.program_id(0)
    seg_lane = lax.broadcasted_iota(jnp.int32, (RB, LANES), 1)
    col = lax.broadcasted_iota(jnp.int32, (RB, VOCAB), 1)
    for k in range(NS):
        yy = ys[k][...]
        t = jnp.sum(
            jnp.where(seg_lane == yy % LANES, gs[k][...], 0.0),
            axis=1,
            keepdims=True,
        )
        vals = xs[k][...]
        m = (vals > t) | ((vals == t) & (col < yy))
        acc_ref[pl.ds((NS * i + k) * RB, RB), :] = jnp.sum(
            m.astype(jnp.int32), axis=1, keepdims=True
        )

    @pl.when(i == NRB // NS - 1)
    def _():
        out_ref[...] = jnp.sum(
            (acc_ref[...] < TOPK).astype(jnp.int32), axis=(0, 1), keepdims=True
        )


def _stream_spec(shape, k):
    return pl.BlockSpec(shape, lambda i, k=k: (NS * i + k, 0))


def _count_hits(y_pred, g, y):
    y2 = y.reshape(B, 1)
    return pl.pallas_call(
        _scan_body,
        grid=(NRB // NS,),
        in_specs=(
            [_stream_spec((RB, LANES), k) for k in range(NS)]
            + [_stream_spec((RB, 1), k) for k in range(NS)]
            + [_stream_spec((RB, VOCAB), k) for k in range(NS)]
        ),
        out_specs=pl.BlockSpec((1, 1), lambda i: (0, 0)),
        out_shape=jax.ShapeDtypeStruct((1, 1), jnp.int32),
        scratch_shapes=[
            pltpu.VMEM((B, 1), jnp.int32),
        ],
    )(*([g] * NS + [y2] * NS + [y_pred] * NS))


def kernel(y_pred, y):
    y32 = y.astype(jnp.int32)
    g = _gather_segments()(y_pred, y32)
    return _count_hits(y_pred, g, y32)[0, 0]
